# baseline (speedup 1.0000x reference)
import functools

import jax
import jax.numpy as jnp
from jax import lax
from jax.experimental import pallas as pl
from jax.experimental.pallas import tpu as pltpu
from jax.experimental.pallas import tpu_sc as plsc

B = 8
T = 8192
Q = 256
T_TILE = 128
N_WORKERS = 32
COLS_PER_W = B * T // N_WORKERS
TILES_PER_W = COLS_PER_W // T_TILE

_mesh = plsc.VectorSubcoreMesh(core_axis_name="c", subcore_axis_name="s")


@functools.partial(
    pl.kernel,
    out_type=jax.ShapeDtypeStruct((B * Q, T), jnp.float32),
    mesh=_mesh,
    compiler_params=pltpu.CompilerParams(needs_layout_passes=False),
    scratch_types=[
        pltpu.VMEM((COLS_PER_W,), jnp.int32),
        pltpu.VMEM((Q, T_TILE), jnp.float32),
        pltpu.VMEM((Q, T_TILE), jnp.float32),
        pltpu.VMEM((Q, T_TILE), jnp.float32),
        pltpu.SemaphoreType.DMA,
        pltpu.SemaphoreType.DMA,
        pltpu.SemaphoreType.DMA,
    ],
)
def _sc_onehot(idx_hbm, zeros_hbm, out_hbm, idx_v, tile0, tile1, tile2,
               sem0, sem1, sem2):
    c = lax.axis_index("c")
    s = lax.axis_index("s")
    wid = s * 2 + c
    base = wid * COLS_PER_W
    b = base // T
    t0 = base % T

    pltpu.sync_copy(idx_hbm.at[pl.ds(base, COLS_PER_W)], idx_v)
    pltpu.sync_copy(zeros_hbm, tile0)
    pltpu.sync_copy(zeros_hbm, tile1)
    pltpu.sync_copy(zeros_hbm, tile2)

    tiles = (tile0, tile1, tile2)
    sems = (sem0, sem1, sem2)
    nbuf = len(tiles)
    lanes = lax.iota(jnp.int32, 16)
    ones = jnp.full((16,), 1.0, jnp.float32)
    zs = jnp.full((16,), 0.0, jnp.float32)

    def scatter(tile, i, vals):
        for j in range(T_TILE // 16):
            rows = idx_v[pl.ds(i * T_TILE + j * 16, 16)]
            plsc.store_scatter(tile, [rows, lanes + (j * 16)], vals)

    def out_slice(i):
        return out_hbm.at[pl.ds(b * Q, Q), pl.ds(t0 + i * T_TILE, T_TILE)]

    for i in range(TILES_PER_W):
        slot = i % nbuf
        tile, sem = tiles[slot], sems[slot]
        if i >= nbuf:
            pltpu.make_async_copy(tile, out_slice(i - nbuf), sem).wait()
            scatter(tile, i - nbuf, zs)
        scatter(tile, i, ones)
        pltpu.make_async_copy(tile, out_slice(i), sem).start()
    for i in range(TILES_PER_W - nbuf, TILES_PER_W):
        pltpu.make_async_copy(tiles[i % nbuf], out_slice(i), sems[i % nbuf]).wait()


def kernel(in_snd_slice, quant_onehot):
    del quant_onehot
    idx = in_snd_slice.astype(jnp.int32).reshape(B * T)
    zeros = jnp.zeros((Q, T_TILE), jnp.float32)
    out = _sc_onehot(idx, zeros)
    return out.reshape(B, Q, T)
